# Initial kernel scaffold; baseline (speedup 1.0000x reference)
import jax
import jax.numpy as jnp
from jax.experimental import pallas as pl


def kernel(src_embedding_0, src_embedding_1, W, b):
    raise NotImplementedError("write your pallas kernel here")


# speedup vs baseline: 7.2894x; 7.2894x over previous
import functools
import jax
import jax.numpy as jnp
from jax.experimental import pallas as pl
from jax.experimental.pallas import tpu as pltpu

_KNN = 10
_HID = 128
_N0 = 4096
_N1 = 4096
_N = _N0 + _N1
_DUMP = _N
_NROWS_ACC = _N + 16
_RB = 256
_NBLK = _N // _RB


def _topk_body(emb_row_ref, emb_col_ref, idx_ref):
    rows = emb_row_ref[...]
    cols = emb_col_ref[...]
    vr = rows / (jnp.sqrt(jnp.sum(rows * rows, axis=1, keepdims=True)) + 1e-12)
    vc = cols / (jnp.sqrt(jnp.sum(cols * cols, axis=1, keepdims=True)) + 1e-12)
    sims = jax.lax.dot_general(
        vr, vc, (((1,), (1,)), ((), ())), preferred_element_type=jnp.float32
    )
    col_iota = jax.lax.broadcasted_iota(jnp.int32, (_RB, _N1), 1)
    col_off = jnp.where(pl.program_id(0) < (_N0 // _RB), _N0, 0).astype(jnp.int32)
    dsts = []
    for _ in range(_KNN):
        m = jnp.max(sims, axis=1)
        eq = sims == m[:, None]
        idx = jnp.min(jnp.where(eq, col_iota, _N1), axis=1)
        sims = jnp.where(col_iota == idx[:, None], -3.0, sims)
        dst = jnp.where(m > 0.0, idx + col_off, _DUMP).astype(jnp.int32)
        dsts.append(dst[:, None])
    pad = jnp.full((_RB, 16 - _KNN), _DUMP, dtype=jnp.int32)
    idx_ref[...] = jnp.concatenate(dsts + [pad], axis=1)


def _topk_edges(emb):
    return pl.pallas_call(
        _topk_body,
        grid=(_NBLK,),
        in_specs=[
            pl.BlockSpec((_RB, _HID), lambda i: (i, 0)),
            pl.BlockSpec((_N1, _HID), lambda i: (1 - i // (_N0 // _RB), 0)),
        ],
        out_specs=pl.BlockSpec((_RB, 16), lambda i: (i, 0)),
        out_shape=jax.ShapeDtypeStruct((_N, 16), jnp.int32),
    )(emb, emb)


def kernel(src_embedding_0, src_embedding_1, W, b):
    emb = jnp.concatenate([src_embedding_0, src_embedding_1], axis=0)
    idx = _topk_edges(emb)

    dst = idx[:, :_KNN].reshape(-1)
    valid = (dst != _DUMP).astype(jnp.float32)
    src = jnp.repeat(jnp.arange(_N), _KNN)
    xw = emb @ W
    deg = jnp.zeros((_N,), jnp.float32).at[dst].add(valid, mode="drop") + 1.0
    dinv = 1.0 / jnp.sqrt(deg)
    y = dinv[:, None] * xw
    acc = jnp.zeros((_N, _HID), jnp.float32).at[dst].add(
        valid[:, None] * y[src], mode="drop"
    )
    out = dinv[:, None] * (acc + y) + b
    return out


# speedup vs baseline: 17.0998x; 2.3458x over previous
import functools
import jax
import jax.numpy as jnp
from jax import lax
from jax.experimental import pallas as pl
from jax.experimental.pallas import tpu as pltpu
from jax.experimental.pallas import tpu_sc as plsc

_KNN = 10
_HID = 128
_N0 = 4096
_N1 = 4096
_N = _N0 + _N1
_DUMP = _N
_NACC = _N + 16
_RB = 256
_NBLK = _N // _RB

_NTILES = 32
_RPT = _N // _NTILES
_CH = 128


def _topk_body(emb_row_ref, emb_col_ref, idxt_ref):
    rows = emb_row_ref[...]
    cols = emb_col_ref[...]
    vr = rows / (jnp.sqrt(jnp.sum(rows * rows, axis=1, keepdims=True)) + 1e-12)
    vc = cols / (jnp.sqrt(jnp.sum(cols * cols, axis=1, keepdims=True)) + 1e-12)
    sims = jax.lax.dot_general(
        vr, vc, (((1,), (1,)), ((), ())), preferred_element_type=jnp.float32
    )
    col_iota = jax.lax.broadcasted_iota(jnp.int32, (_RB, _N1), 1)
    col_off = jnp.where(pl.program_id(0) < (_N0 // _RB), _N0, 0).astype(jnp.int32)
    dsts = []
    for _ in range(_KNN):
        m = jnp.max(sims, axis=1)
        eq = sims == m[:, None]
        idx = jnp.min(jnp.where(eq, col_iota, _N1), axis=1)
        sims = jnp.where(col_iota == idx[:, None], -3.0, sims)
        dst = jnp.where(m > 0.0, idx + col_off, _DUMP).astype(jnp.int32)
        dsts.append(dst[:, None])
    arr = jnp.concatenate(dsts, axis=1)
    idxt_ref[...] = arr.T


def _topk_edges(emb):
    return pl.pallas_call(
        _topk_body,
        grid=(_NBLK,),
        in_specs=[
            pl.BlockSpec((_RB, _HID), lambda i: (i, 0)),
            pl.BlockSpec((_N1, _HID), lambda i: (1 - i // (_N0 // _RB), 0)),
        ],
        out_specs=pl.BlockSpec((_KNN, _RB), lambda i: (0, i)),
        out_shape=jax.ShapeDtypeStruct((_KNN, _N), jnp.int32),
    )(emb, emb)


_MESH = plsc.VectorSubcoreMesh(core_axis_name="c", subcore_axis_name="s")


@functools.partial(
    pl.kernel,
    out_type=jax.ShapeDtypeStruct((2, _NACC, _HID), jnp.float32),
    mesh=_MESH,
    scratch_types=[
        pltpu.VMEM((_CH,), jnp.int32),
        pltpu.VMEM((_CH,), jnp.int32),
        pltpu.VMEM((_CH, _HID), jnp.float32),
        pltpu.VMEM_SHARED((_NACC, _HID), jnp.float32),
    ],
)
def _deg_kernel(idxt_hbm, ones_hbm, zeros_hbm, out_hbm,
                colidx_a, colidx_b, ones_v, acc_sh):
    c = lax.axis_index("c")
    s = lax.axis_index("s")
    wid = c * 16 + s
    r0 = wid * _RPT

    @pl.when(s == 0)
    def _init():
        pltpu.sync_copy(zeros_hbm, acc_sh)

    pltpu.sync_copy(ones_hbm, ones_v)
    plsc.subcore_barrier()
    for j in range(_KNN):
        pltpu.sync_copy(idxt_hbm.at[j, pl.ds(r0, _CH)], colidx_a)
        pltpu.sync_copy(idxt_hbm.at[j, pl.ds(r0 + _CH, _CH)], colidx_b)
        pltpu.sync_copy(ones_v, acc_sh.at[colidx_a], add=True)
        pltpu.sync_copy(ones_v, acc_sh.at[colidx_b], add=True)
    plsc.subcore_barrier()

    @pl.when(s == 0)
    def _dump():
        pltpu.sync_copy(acc_sh, out_hbm.at[c])


@functools.partial(
    pl.kernel,
    out_type=jax.ShapeDtypeStruct((2, _NACC, _HID), jnp.float32),
    mesh=_MESH,
    scratch_types=[
        pltpu.VMEM((_CH,), jnp.int32),
        pltpu.VMEM((_CH,), jnp.int32),
        pltpu.VMEM((_RPT, _HID), jnp.float32),
        pltpu.VMEM_SHARED((_NACC, _HID), jnp.float32),
    ],
)
def _scatter_kernel(y_hbm, idxt_hbm, zeros_hbm, out_hbm,
                    colidx_a, colidx_b, y_slab, acc_sh):
    c = lax.axis_index("c")
    s = lax.axis_index("s")
    wid = c * 16 + s
    r0 = wid * _RPT

    @pl.when(s == 0)
    def _init():
        pltpu.sync_copy(zeros_hbm, acc_sh)

    pltpu.sync_copy(y_hbm.at[pl.ds(r0, _RPT)], y_slab)
    plsc.subcore_barrier()
    for j in range(_KNN):
        pltpu.sync_copy(idxt_hbm.at[j, pl.ds(r0, _CH)], colidx_a)
        pltpu.sync_copy(idxt_hbm.at[j, pl.ds(r0 + _CH, _CH)], colidx_b)
        pltpu.sync_copy(y_slab.at[pl.ds(0, _CH)], acc_sh.at[colidx_a], add=True)
        pltpu.sync_copy(y_slab.at[pl.ds(_CH, _CH)], acc_sh.at[colidx_b], add=True)
    plsc.subcore_barrier()

    @pl.when(s == 0)
    def _dump():
        pltpu.sync_copy(acc_sh, out_hbm.at[c])


def _y_body(emb_ref, w_ref, deg_ref, y_ref):
    deg = deg_ref[0, :, 0] + deg_ref[1, :, 0] + 1.0
    dinv = 1.0 / jnp.sqrt(deg)
    xw = jax.lax.dot_general(
        emb_ref[...], w_ref[...], (((1,), (0,)), ((), ())),
        preferred_element_type=jnp.float32,
    )
    y_ref[...] = dinv[:, None] * xw


def _y_kernel(emb, W, deg_parts):
    blk = 1024
    return pl.pallas_call(
        _y_body,
        grid=(_N // blk,),
        in_specs=[
            pl.BlockSpec((blk, _HID), lambda i: (i, 0)),
            pl.BlockSpec((_HID, _HID), lambda i: (0, 0)),
            pl.BlockSpec((2, blk, _HID), lambda i: (0, i, 0)),
        ],
        out_specs=pl.BlockSpec((blk, _HID), lambda i: (i, 0)),
        out_shape=jax.ShapeDtypeStruct((_N, _HID), jnp.float32),
    )(emb, W, deg_parts)


def _out_body(acc_ref, deg_ref, y_ref, b_ref, out_ref):
    deg = deg_ref[0, :, 0] + deg_ref[1, :, 0] + 1.0
    dinv = 1.0 / jnp.sqrt(deg)
    out_ref[...] = dinv[:, None] * (acc_ref[0] + acc_ref[1] + y_ref[...]) + b_ref[...]


def _out_kernel(acc_parts, deg_parts, y, b2):
    blk = 1024
    return pl.pallas_call(
        _out_body,
        grid=(_N // blk,),
        in_specs=[
            pl.BlockSpec((2, blk, _HID), lambda i: (0, i, 0)),
            pl.BlockSpec((2, blk, _HID), lambda i: (0, i, 0)),
            pl.BlockSpec((blk, _HID), lambda i: (i, 0)),
            pl.BlockSpec((1, _HID), lambda i: (0, 0)),
        ],
        out_specs=pl.BlockSpec((blk, _HID), lambda i: (i, 0)),
        out_shape=jax.ShapeDtypeStruct((_N, _HID), jnp.float32),
    )(acc_parts, deg_parts, y, b2)


def kernel(src_embedding_0, src_embedding_1, W, b):
    emb = jnp.concatenate([src_embedding_0, src_embedding_1], axis=0)
    idxt = _topk_edges(emb)
    z128 = jnp.zeros((_NACC, _HID), jnp.float32)
    lane = jnp.arange(_HID)
    ones_rows = jnp.broadcast_to((lane == 0).astype(jnp.float32), (_CH, _HID))
    deg_parts = _deg_kernel(idxt, ones_rows, z128)
    y = _y_kernel(emb, W, deg_parts)
    acc_parts = _scatter_kernel(y, idxt, z128)
    return _out_kernel(acc_parts, deg_parts, y, b.reshape(1, _HID))


# speedup vs baseline: 18.7863x; 1.0986x over previous
import functools
import jax
import jax.numpy as jnp
from jax import lax
from jax.experimental import pallas as pl
from jax.experimental.pallas import tpu as pltpu
from jax.experimental.pallas import tpu_sc as plsc

_KNN = 10
_HID = 128
_N0 = 4096
_N1 = 4096
_N = _N0 + _N1
_DUMP = _N
_NACC = _N + 16
_RB = 256
_NBLK = _N // _RB

_NTILES = 32
_RPT = _N // _NTILES
_CH = 128


def _topk_body(emb_row_ref, emb_col_ref, idxt_ref):
    rows = emb_row_ref[...]
    cols = emb_col_ref[...]
    vr = rows / (jnp.sqrt(jnp.sum(rows * rows, axis=1, keepdims=True)) + 1e-12)
    vc = cols / (jnp.sqrt(jnp.sum(cols * cols, axis=1, keepdims=True)) + 1e-12)
    sims = jax.lax.dot_general(
        vr, vc, (((1,), (1,)), ((), ())), preferred_element_type=jnp.float32
    )
    col_iota = jax.lax.broadcasted_iota(jnp.int32, (_RB, _N1), 1)
    col_off = jnp.where(pl.program_id(0) < (_N0 // _RB), _N0, 0).astype(jnp.int32)
    npos = jnp.sum((sims > 0.0).astype(jnp.float32), axis=1)
    dsts = []
    for j in range(_KNN):
        am = jnp.argmax(sims, axis=1).astype(jnp.int32)
        eqpos = col_iota == am[:, None]
        sims = jnp.where(eqpos, -3.0, sims)
        dst = jnp.where(npos > j, am + col_off, _DUMP).astype(jnp.int32)
        dsts.append(dst[:, None])
    arr = jnp.concatenate(dsts, axis=1)
    idxt_ref[...] = arr.T


def _topk_edges(emb):
    return pl.pallas_call(
        _topk_body,
        grid=(_NBLK,),
        in_specs=[
            pl.BlockSpec((_RB, _HID), lambda i: (i, 0)),
            pl.BlockSpec((_N1, _HID), lambda i: (1 - i // (_N0 // _RB), 0)),
        ],
        out_specs=pl.BlockSpec((_KNN, _RB), lambda i: (0, i)),
        out_shape=jax.ShapeDtypeStruct((_KNN, _N), jnp.int32),
    )(emb, emb)


_MESH = plsc.VectorSubcoreMesh(core_axis_name="c", subcore_axis_name="s")


@functools.partial(
    pl.kernel,
    out_type=jax.ShapeDtypeStruct((2, _NACC, _HID), jnp.float32),
    mesh=_MESH,
    scratch_types=[
        pltpu.VMEM((_CH,), jnp.int32),
        pltpu.VMEM((_CH,), jnp.int32),
        pltpu.VMEM((_CH, _HID), jnp.float32),
        pltpu.VMEM_SHARED((_NACC, _HID), jnp.float32),
    ],
)
def _deg_kernel(idxt_hbm, ones_hbm, zeros_hbm, out_hbm,
                colidx_a, colidx_b, ones_v, acc_sh):
    c = lax.axis_index("c")
    s = lax.axis_index("s")
    wid = c * 16 + s
    r0 = wid * _RPT

    @pl.when(s == 0)
    def _init():
        pltpu.sync_copy(zeros_hbm, acc_sh)

    pltpu.sync_copy(ones_hbm, ones_v)
    plsc.subcore_barrier()
    for j in range(_KNN):
        pltpu.sync_copy(idxt_hbm.at[j, pl.ds(r0, _CH)], colidx_a)
        pltpu.sync_copy(idxt_hbm.at[j, pl.ds(r0 + _CH, _CH)], colidx_b)
        pltpu.sync_copy(ones_v, acc_sh.at[colidx_a], add=True)
        pltpu.sync_copy(ones_v, acc_sh.at[colidx_b], add=True)
    plsc.subcore_barrier()

    @pl.when(s == 0)
    def _dump():
        pltpu.sync_copy(acc_sh, out_hbm.at[c])


@functools.partial(
    pl.kernel,
    out_type=jax.ShapeDtypeStruct((2, _NACC, _HID), jnp.float32),
    mesh=_MESH,
    scratch_types=[
        pltpu.VMEM((_CH,), jnp.int32),
        pltpu.VMEM((_CH,), jnp.int32),
        pltpu.VMEM((_RPT, _HID), jnp.float32),
        pltpu.VMEM_SHARED((_NACC, _HID), jnp.float32),
    ],
)
def _scatter_kernel(y_hbm, idxt_hbm, zeros_hbm, out_hbm,
                    colidx_a, colidx_b, y_slab, acc_sh):
    c = lax.axis_index("c")
    s = lax.axis_index("s")
    wid = c * 16 + s
    r0 = wid * _RPT

    @pl.when(s == 0)
    def _init():
        pltpu.sync_copy(zeros_hbm, acc_sh)

    pltpu.sync_copy(y_hbm.at[pl.ds(r0, _RPT)], y_slab)
    plsc.subcore_barrier()
    for j in range(_KNN):
        pltpu.sync_copy(idxt_hbm.at[j, pl.ds(r0, _CH)], colidx_a)
        pltpu.sync_copy(idxt_hbm.at[j, pl.ds(r0 + _CH, _CH)], colidx_b)
        pltpu.sync_copy(y_slab.at[pl.ds(0, _CH)], acc_sh.at[colidx_a], add=True)
        pltpu.sync_copy(y_slab.at[pl.ds(_CH, _CH)], acc_sh.at[colidx_b], add=True)
    plsc.subcore_barrier()

    @pl.when(s == 0)
    def _dump():
        pltpu.sync_copy(acc_sh, out_hbm.at[c])


def _y_body(emb_ref, w_ref, deg_ref, y_ref):
    deg = deg_ref[0, :, 0] + deg_ref[1, :, 0] + 1.0
    dinv = 1.0 / jnp.sqrt(deg)
    xw = jax.lax.dot_general(
        emb_ref[...], w_ref[...], (((1,), (0,)), ((), ())),
        preferred_element_type=jnp.float32,
    )
    y_ref[...] = dinv[:, None] * xw


def _y_kernel(emb, W, deg_parts):
    blk = 1024
    return pl.pallas_call(
        _y_body,
        grid=(_N // blk,),
        in_specs=[
            pl.BlockSpec((blk, _HID), lambda i: (i, 0)),
            pl.BlockSpec((_HID, _HID), lambda i: (0, 0)),
            pl.BlockSpec((2, blk, _HID), lambda i: (0, i, 0)),
        ],
        out_specs=pl.BlockSpec((blk, _HID), lambda i: (i, 0)),
        out_shape=jax.ShapeDtypeStruct((_N, _HID), jnp.float32),
    )(emb, W, deg_parts)


def _out_body(acc_ref, deg_ref, y_ref, b_ref, out_ref):
    deg = deg_ref[0, :, 0] + deg_ref[1, :, 0] + 1.0
    dinv = 1.0 / jnp.sqrt(deg)
    out_ref[...] = dinv[:, None] * (acc_ref[0] + acc_ref[1] + y_ref[...]) + b_ref[...]


def _out_kernel(acc_parts, deg_parts, y, b2):
    blk = 1024
    return pl.pallas_call(
        _out_body,
        grid=(_N // blk,),
        in_specs=[
            pl.BlockSpec((2, blk, _HID), lambda i: (0, i, 0)),
            pl.BlockSpec((2, blk, _HID), lambda i: (0, i, 0)),
            pl.BlockSpec((blk, _HID), lambda i: (i, 0)),
            pl.BlockSpec((1, _HID), lambda i: (0, 0)),
        ],
        out_specs=pl.BlockSpec((blk, _HID), lambda i: (i, 0)),
        out_shape=jax.ShapeDtypeStruct((_N, _HID), jnp.float32),
    )(acc_parts, deg_parts, y, b2)


def kernel(src_embedding_0, src_embedding_1, W, b):
    emb = jnp.concatenate([src_embedding_0, src_embedding_1], axis=0)
    idxt = _topk_edges(emb)
    z128 = jnp.zeros((_NACC, _HID), jnp.float32)
    lane = jnp.arange(_HID)
    ones_rows = jnp.broadcast_to((lane == 0).astype(jnp.float32), (_CH, _HID))
    deg_parts = _deg_kernel(idxt, ones_rows, z128)
    y = _y_kernel(emb, W, deg_parts)
    acc_parts = _scatter_kernel(y, idxt, z128)
    return _out_kernel(acc_parts, deg_parts, y, b.reshape(1, _HID))
